# baseline (speedup 1.0000x reference)
import functools

import jax
import jax.numpy as jnp
from jax import lax
from jax.experimental import pallas as pl
from jax.experimental.pallas import tpu as pltpu
from jax.experimental.pallas import tpu_sc as plsc

_MASK_RATIO = 0.15
_ROW_BLOCK = 512
_LANES = 16


def _sc_mask_kernel(t, idx_pad, idx_hbm, mask_hbm, mask_v, idx_v):
    wid = lax.axis_index("s") * 2 + lax.axis_index("c")

    @pl.when(wid == 0)
    def _():
        pltpu.sync_copy(idx_hbm, idx_v)
        ones = jnp.ones((_LANES,), jnp.float32)
        for i in range((t + _LANES) // _LANES):
            mask_v[pl.ds(i * _LANES, _LANES)] = ones
        zeros = jnp.zeros((_LANES,), jnp.float32)
        for i in range(idx_pad // _LANES):
            iv = idx_v[pl.ds(i * _LANES, _LANES)]
            plsc.store_scatter(mask_v, [iv], zeros)
        pltpu.sync_copy(mask_v.at[pl.ds(0, t)], mask_hbm)


def _mask_mul_kernel(mask_ref, x_ref, o_ref):
    o_ref[...] = x_ref[...] * mask_ref[...]


def kernel(x):
    B, C, T = x.shape
    n_mask = int(T * _MASK_RATIO)
    if n_mask <= 0:
        return x

    key = jax.random.fold_in(jax.random.key(0), 1)
    mask_indices = jax.random.permutation(key, T)[:n_mask].astype(jnp.int32)

    idx_pad = ((n_mask + _LANES - 1) // _LANES) * _LANES
    idx1d = jnp.concatenate(
        [mask_indices, jnp.full((idx_pad - n_mask,), T, jnp.int32)]
    )

    mesh = plsc.VectorSubcoreMesh(core_axis_name="c", subcore_axis_name="s")
    sc_build = functools.partial(
        pl.kernel,
        mesh=mesh,
        out_type=jax.ShapeDtypeStruct((T,), jnp.float32),
        scratch_types=[
            pltpu.VMEM((T + _LANES,), jnp.float32),
            pltpu.VMEM((idx_pad,), jnp.int32),
        ],
        compiler_params=pltpu.CompilerParams(needs_layout_passes=False),
    )(functools.partial(_sc_mask_kernel, T, idx_pad))
    time_mask = sc_build(idx1d).reshape(1, T)

    rows = B * C
    xr = x.reshape(rows, T)
    grid = (rows // _ROW_BLOCK,)

    out = pl.pallas_call(
        _mask_mul_kernel,
        grid=grid,
        in_specs=[
            pl.BlockSpec((1, T), lambda i: (0, 0)),
            pl.BlockSpec((_ROW_BLOCK, T), lambda i: (i, 0)),
        ],
        out_specs=pl.BlockSpec((_ROW_BLOCK, T), lambda i: (i, 0)),
        out_shape=jax.ShapeDtypeStruct((rows, T), x.dtype),
        compiler_params=pltpu.CompilerParams(
            dimension_semantics=("parallel",),
        ),
    )(time_mask, xr)
    return out.reshape(B, C, T)
